# baseline (speedup 1.0000x reference)
import functools

import jax
import jax.numpy as jnp
from jax import lax
from jax.experimental import pallas as pl
from jax.experimental.pallas import tpu as pltpu
from jax.experimental.pallas import tpu_sc as plsc

NC = 2
NS = 16
NW = NC * NS
BATCH = 16384
D = 32
BPW = BATCH // NW
CHUNK = 128
NCHUNK = BPW // CHUNK
GROUPS = BPW // 16


def _sc_body(uidx_hbm, iidx_hbm, utab_hbm, itab_hbm, wb_hbm, out_hbm,
             uidx_v, iidx_v, ubuf, ibuf, wb_v, ptile, obuf, sem):
    wid = lax.axis_index("s") * NC + lax.axis_index("c")
    base = wid * BPW

    pltpu.sync_copy(uidx_hbm.at[wid], uidx_v)
    pltpu.sync_copy(iidx_hbm.at[wid], iidx_v)
    pltpu.sync_copy(wb_hbm, wb_v)

    copies = []
    for j in range(NCHUNK):
        copies.append(pltpu.async_copy(
            utab_hbm.at[uidx_v.at[j]], ubuf.at[pl.ds(j * CHUNK, CHUNK)], sem))
        copies.append(pltpu.async_copy(
            itab_hbm.at[iidx_v.at[j]], ibuf.at[pl.ds(j * CHUNK, CHUNK)], sem))
    for c in copies:
        c.wait()

    w0 = wb_v[pl.ds(0, 16)]
    w1 = wb_v[pl.ds(16, 16)]
    bv = wb_v[pl.ds(32, 16)]
    rowids = lax.iota(jnp.int32, 16)

    def group_body(g, carry):
        r0 = g * 16
        for j in range(16):
            r = r0 + j
            p = (ubuf[r, pl.ds(0, 16)] * ibuf[r, pl.ds(0, 16)] * w0
                 + ubuf[r, pl.ds(16, 16)] * ibuf[r, pl.ds(16, 16)] * w1)
            ptile[j, :] = p
        acc = bv
        for c in range(16):
            colids = jnp.full((16,), c, jnp.int32)
            acc = acc + plsc.load_gather(ptile, [rowids, colids])
        score = 1.0 / (1.0 + jnp.exp(-acc))
        obuf[pl.ds(r0, 16)] = score
        return carry

    lax.fori_loop(0, GROUPS, group_body, 0)

    pltpu.sync_copy(obuf, out_hbm.at[pl.ds(base, BPW)])


@jax.jit
def kernel(user_indices, item_indices, user_table, item_table, W, b):
    uidx = user_indices.astype(jnp.int32).reshape(NW, NCHUNK, CHUNK)
    iidx = item_indices.astype(jnp.int32).reshape(NW, NCHUNK, CHUNK)
    wb = jnp.concatenate(
        [W.reshape(D).astype(jnp.float32),
         jnp.broadcast_to(b.astype(jnp.float32).reshape(1), (16,))])

    mesh = plsc.VectorSubcoreMesh(
        core_axis_name="c", subcore_axis_name="s",
        num_cores=NC, num_subcores=NS)
    fn = pl.kernel(
        _sc_body,
        out_type=jax.ShapeDtypeStruct((BATCH,), jnp.float32),
        mesh=mesh,
        compiler_params=pltpu.CompilerParams(
            needs_layout_passes=False, use_tc_tiling_on_sc=False),
        scratch_types=[
            pltpu.VMEM((NCHUNK, CHUNK), jnp.int32),
            pltpu.VMEM((NCHUNK, CHUNK), jnp.int32),
            pltpu.VMEM((BPW, D), jnp.float32),
            pltpu.VMEM((BPW, D), jnp.float32),
            pltpu.VMEM((48,), jnp.float32),
            pltpu.VMEM((16, 16), jnp.float32),
            pltpu.VMEM((BPW,), jnp.float32),
            pltpu.SemaphoreType.DMA,
        ],
    )
    return fn(uidx, iidx, user_table, item_table, wb)


# speedup vs baseline: 1.5603x; 1.5603x over previous
import jax
import jax.numpy as jnp
from jax import lax
from jax.experimental import pallas as pl
from jax.experimental.pallas import tpu as pltpu
from jax.experimental.pallas import tpu_sc as plsc

NC = 2
NS = 16
NW = NC * NS
BATCH = 16384
D = 32
BPW = BATCH // NW
PASS = 256
NPASS = BPW // PASS
PGROUPS = PASS // 16


def _sc_body(uidx_hbm, iidx_hbm, utab_hbm, itab_hbm, wb_hbm, out_hbm,
             uidx_v, iidx_v, ubuf, ibuf, wb_v, ptile, obuf, sem):
    wid = lax.axis_index("s") * NC + lax.axis_index("c")
    base = wid * BPW

    pltpu.sync_copy(uidx_hbm.at[pl.ds(base, BPW)], uidx_v)
    pltpu.sync_copy(iidx_hbm.at[pl.ds(base, BPW)], iidx_v)
    pltpu.sync_copy(wb_hbm, wb_v)

    w0 = wb_v[pl.ds(0, 16)]
    w1 = wb_v[pl.ds(16, 16)]
    bv = wb_v[pl.ds(32, 16)]
    rowids = lax.iota(jnp.int32, 16) * 16

    for p in range(NPASS):
        def fire_body(g, carry):
            off = g * 16
            iv_u = uidx_v[pl.ds(p * PASS + off, 16)]
            iv_i = iidx_v[pl.ds(p * PASS + off, 16)]
            for j in range(16):
                pltpu.async_copy(utab_hbm.at[iv_u[j]], ubuf.at[off + j], sem)
                pltpu.async_copy(itab_hbm.at[iv_i[j]], ibuf.at[off + j], sem)
            return carry

        lax.fori_loop(0, PGROUPS, fire_body, 0)

        pltpu.make_async_copy(utab_hbm.at[pl.ds(0, PASS)], ubuf, sem).wait()
        pltpu.make_async_copy(itab_hbm.at[pl.ds(0, PASS)], ibuf, sem).wait()

        def group_body(g, carry):
            r0 = g * 16
            for j in range(16):
                r = r0 + j
                p_ = (ubuf[r, pl.ds(0, 16)] * ibuf[r, pl.ds(0, 16)] * w0
                      + ubuf[r, pl.ds(16, 16)] * ibuf[r, pl.ds(16, 16)] * w1)
                ptile[pl.ds(j * 16, 16)] = p_
            acc = bv
            for c in range(16):
                colids = rowids + c
                acc = acc + plsc.load_gather(ptile, [colids])
            score = 1.0 / (1.0 + jnp.exp(-acc))
            obuf[pl.ds(p * PASS + r0, 16)] = score
            return carry

        lax.fori_loop(0, PGROUPS, group_body, 0)

    pltpu.sync_copy(obuf, out_hbm.at[pl.ds(base, BPW)])


@jax.jit
def kernel(user_indices, item_indices, user_table, item_table, W, b):
    uidx = user_indices.astype(jnp.int32)
    iidx = item_indices.astype(jnp.int32)
    wb = jnp.concatenate(
        [W.reshape(D).astype(jnp.float32),
         jnp.broadcast_to(b.astype(jnp.float32).reshape(1), (16,))])

    mesh = plsc.VectorSubcoreMesh(
        core_axis_name="c", subcore_axis_name="s",
        num_cores=NC, num_subcores=NS)
    fn = pl.kernel(
        _sc_body,
        out_type=jax.ShapeDtypeStruct((BATCH,), jnp.float32),
        mesh=mesh,
        compiler_params=pltpu.CompilerParams(needs_layout_passes=False),
        scratch_types=[
            pltpu.VMEM((BPW,), jnp.int32),
            pltpu.VMEM((BPW,), jnp.int32),
            pltpu.VMEM((PASS, D), jnp.float32),
            pltpu.VMEM((PASS, D), jnp.float32),
            pltpu.VMEM((48,), jnp.float32),
            pltpu.VMEM((256,), jnp.float32),
            pltpu.VMEM((BPW,), jnp.float32),
            pltpu.SemaphoreType.DMA,
        ],
    )
    return fn(uidx, iidx, user_table, item_table, wb)


# speedup vs baseline: 1.5683x; 1.0051x over previous
import jax
import jax.numpy as jnp
from jax import lax
from jax.experimental import pallas as pl
from jax.experimental.pallas import tpu as pltpu
from jax.experimental.pallas import tpu_sc as plsc

NC = 2
NS = 16
NW = NC * NS
BATCH = 16384
D = 32
BPW = BATCH // NW
PASS = 128
NPASS = BPW // PASS
PGROUPS = PASS // 16


def _sc_body(uidx_hbm, iidx_hbm, utab_hbm, itab_hbm, wb_hbm, out_hbm,
             uidx_v, iidx_v, ubufs, ibufs, wb_v, ptile, obuf, sems):
    wid = lax.axis_index("s") * NC + lax.axis_index("c")
    base = wid * BPW

    pltpu.sync_copy(uidx_hbm.at[pl.ds(base, BPW)], uidx_v)
    pltpu.sync_copy(iidx_hbm.at[pl.ds(base, BPW)], iidx_v)
    pltpu.sync_copy(wb_hbm, wb_v)

    w0 = wb_v[pl.ds(0, 16)]
    w1 = wb_v[pl.ds(16, 16)]
    bv = wb_v[pl.ds(32, 16)]
    rowids = lax.iota(jnp.int32, 16) * 16

    def fire(p, buf_slot):
        ubuf, ibuf, sem = ubufs[buf_slot], ibufs[buf_slot], sems[buf_slot]

        def fire_body(g, carry):
            off = g * 16
            iv_u = uidx_v[pl.ds(p * PASS + off, 16)]
            iv_i = iidx_v[pl.ds(p * PASS + off, 16)]
            for j in range(16):
                pltpu.async_copy(utab_hbm.at[iv_u[j]], ubuf.at[off + j], sem)
                pltpu.async_copy(itab_hbm.at[iv_i[j]], ibuf.at[off + j], sem)
            return carry

        lax.fori_loop(0, PGROUPS, fire_body, 0)

    def drain(buf_slot):
        pltpu.make_async_copy(
            utab_hbm.at[pl.ds(0, PASS)], ubufs[buf_slot], sems[buf_slot]
        ).wait()
        pltpu.make_async_copy(
            itab_hbm.at[pl.ds(0, PASS)], ibufs[buf_slot], sems[buf_slot]
        ).wait()

    def compute(p, buf_slot):
        ubuf, ibuf = ubufs[buf_slot], ibufs[buf_slot]

        def group_body(g, carry):
            r0 = g * 16
            for j in range(16):
                r = r0 + j
                p_ = (ubuf[r, pl.ds(0, 16)] * ibuf[r, pl.ds(0, 16)] * w0
                      + ubuf[r, pl.ds(16, 16)] * ibuf[r, pl.ds(16, 16)] * w1)
                ptile[pl.ds(j * 16, 16)] = p_
            acc = bv
            for c in range(16):
                colids = rowids + c
                acc = acc + plsc.load_gather(ptile, [colids])
            score = 1.0 / (1.0 + jnp.exp(-acc))
            obuf[pl.ds(p * PASS + r0, 16)] = score
            return carry

        lax.fori_loop(0, PGROUPS, group_body, 0)

    fire(0, 0)
    for p in range(NPASS):
        if p + 1 < NPASS:
            fire(p + 1, (p + 1) % 2)
        drain(p % 2)
        compute(p, p % 2)

    pltpu.sync_copy(obuf, out_hbm.at[pl.ds(base, BPW)])


@jax.jit
def kernel(user_indices, item_indices, user_table, item_table, W, b):
    uidx = user_indices.astype(jnp.int32)
    iidx = item_indices.astype(jnp.int32)
    wb = jnp.concatenate(
        [W.reshape(D).astype(jnp.float32),
         jnp.broadcast_to(b.astype(jnp.float32).reshape(1), (16,))])

    mesh = plsc.VectorSubcoreMesh(
        core_axis_name="c", subcore_axis_name="s",
        num_cores=NC, num_subcores=NS)
    fn = pl.kernel(
        _sc_body,
        out_type=jax.ShapeDtypeStruct((BATCH,), jnp.float32),
        mesh=mesh,
        compiler_params=pltpu.CompilerParams(needs_layout_passes=False),
        scratch_types=[
            pltpu.VMEM((BPW,), jnp.int32),
            pltpu.VMEM((BPW,), jnp.int32),
            [pltpu.VMEM((PASS, D), jnp.float32) for _ in range(2)],
            [pltpu.VMEM((PASS, D), jnp.float32) for _ in range(2)],
            pltpu.VMEM((48,), jnp.float32),
            pltpu.VMEM((256,), jnp.float32),
            pltpu.VMEM((BPW,), jnp.float32),
            [pltpu.SemaphoreType.DMA for _ in range(2)],
        ],
    )
    return fn(uidx, iidx, user_table, item_table, wb)
